# Initial kernel scaffold; baseline (speedup 1.0000x reference)
import jax
import jax.numpy as jnp
from jax.experimental import pallas as pl


def kernel(num_samples, y, loc, log_scale):
    raise NotImplementedError("write your pallas kernel here")


# speedup vs baseline: 2.5717x; 2.5717x over previous
import functools

import numpy as np
import jax
import jax.numpy as jnp
from jax import lax
from jax.experimental import pallas as pl
from jax.experimental.pallas import tpu as pltpu
from jax.experimental.pallas import tpu_sc as plsc

_LANE = 16


def _lane_shuffle(x, idx):
    return lax.gather(
        x, idx[:, None],
        lax.GatherDimensionNumbers(offset_dims=(), collapsed_slice_dims=(0,),
                                   start_index_map=(0,)),
        slice_sizes=(1,), mode=lax.GatherScatterMode.PROMISE_IN_BOUNDS)


def _sc_kernel_body(nc, b_per_w, chunk, d, c_log,
                    y_hbm, locT_hbm, lsT_hbm, eps_hbm,
                    z_hbm, logp_hbm,
                    idx_v, locr_v, lsr_v, eps_v, logp_v, sem):
    wid = lax.axis_index("s") * nc + lax.axis_index("c")
    nvec = d // _LANE

    for c in range(b_per_w // chunk):
        base = wid * b_per_w + c * chunk
        pltpu.sync_copy(y_hbm.at[pl.ds(base, chunk)], idx_v)
        cp_loc = pltpu.async_copy(locT_hbm.at[idx_v], locr_v, sem)
        cp_ls = pltpu.async_copy(lsT_hbm.at[idx_v], lsr_v, sem)
        cp_eps = pltpu.async_copy(eps_hbm.at[pl.ds(base, chunk), :], eps_v, sem)
        cp_loc.wait()
        cp_ls.wait()
        cp_eps.wait()

        lane = lax.iota(jnp.int32, _LANE)

        def group_body(gr, _):
            s0 = gr * _LANE
            res = jnp.zeros((_LANE,), jnp.float32)
            for j in range(_LANE):
                s = s0 + j
                acc = jnp.zeros((_LANE,), jnp.float32)
                for f in range(nvec):
                    sl = pl.ds(f * _LANE, _LANE)
                    e = eps_v.at[s][sl]
                    g = lsr_v.at[s][sl]
                    l = locr_v.at[s][sl]
                    locr_v.at[s][sl] = l + jnp.exp(g) * e
                    acc = acc + g + 0.5 * (e * e)
                for k in (8, 4, 2, 1):
                    acc = acc + _lane_shuffle(acc, lane ^ k)
                res = jnp.where(lane == j, acc, res)
            logp_v[pl.ds(s0, _LANE)] = c_log - res
            return 0

        lax.fori_loop(0, chunk // _LANE, group_body, 0)

        pltpu.sync_copy(locr_v, z_hbm.at[pl.ds(base, chunk), :])
        pltpu.sync_copy(logp_v, logp_hbm.at[pl.ds(base, chunk)])


def kernel(num_samples, y, loc, log_scale):
    del num_samples
    d, ncls = loc.shape
    batch = y.shape[0]

    mesh = plsc.VectorSubcoreMesh(core_axis_name="c", subcore_axis_name="s")
    nw = mesh.num_cores * mesh.num_subcores
    b_per_w = batch // nw
    chunk = 128

    locT = loc.T
    lsT = log_scale.T
    eps = jax.random.normal(jax.random.key(42), (batch, d), dtype=loc.dtype)
    c_log = np.float32(-0.5 * d * np.log(2.0 * np.pi))

    body = functools.partial(_sc_kernel_body, mesh.num_cores, b_per_w, chunk,
                             d, c_log)
    z, logp = pl.kernel(
        body,
        out_type=(
            jax.ShapeDtypeStruct((batch, d), jnp.float32),
            jax.ShapeDtypeStruct((batch,), jnp.float32),
        ),
        mesh=mesh,
        scratch_types=(
            pltpu.VMEM((chunk,), jnp.int32),
            pltpu.VMEM((chunk, d), jnp.float32),
            pltpu.VMEM((chunk, d), jnp.float32),
            pltpu.VMEM((chunk, d), jnp.float32),
            pltpu.VMEM((chunk,), jnp.float32),
            pltpu.SemaphoreType.DMA,
        ),
    )(y, locT, lsT, eps)
    return (z, logp)
